# Initial kernel scaffold; baseline (speedup 1.0000x reference)
import functools

import jax
import jax.numpy as jnp
from jax import lax
from jax.experimental import pallas as pl
from jax.experimental.pallas import tpu as pltpu

N = 10000
E = 50000
H = 64
G = 50
FD = 128
START = 0.0
STOP = 5.0

NW = 32
CHUNK = 128
NCHUNK = 13
E_PAD = NW * NCHUNK * CHUNK
BE = 256
N_BLK = 1000

_COEFF = -0.5 / ((STOP - START) / (G - 1)) ** 2
_OFF_SCALE = (STOP - START) / (G - 1)
_LOG2 = 0.6931471805599453


def _ssp(v):
    return jnp.maximum(v, 0.0) + jnp.log1p(jnp.exp(-jnp.abs(v))) - _LOG2


def _msg_body(ea_ref, xj_ref, w1_ref, b1_ref, w2_ref, b2_ref, w3_ref, b3_ref,
              s_ref, msg_ref):
    ea = ea_ref[...]
    off = lax.broadcasted_iota(jnp.float32, (BE, 64), 1) * _OFF_SCALE
    d = ea - off
    rbf = jnp.exp(_COEFF * d * d)
    h = _ssp(jnp.dot(rbf, w1_ref[...], preferred_element_type=jnp.float32)
             + b1_ref[...])
    h = _ssp(jnp.dot(h, w2_ref[...], preferred_element_type=jnp.float32)
             + b2_ref[...])
    f3 = jnp.dot(h, w3_ref[...], preferred_element_type=jnp.float32) + b3_ref[...]
    xt = jnp.tile(xj_ref[...], (1, H))
    prod = f3 * xt
    msg_ref[...] = jnp.dot(prod, s_ref[...], preferred_element_type=jnp.float32)


def _msg_pallas(ea_pad, xj, w1p, b1r, w2, b2r, w3, b3r, smat):
    grid = E_PAD // BE
    return pl.pallas_call(
        _msg_body,
        grid=grid,
        in_specs=[
            pl.BlockSpec((BE, 1), lambda i: (i, 0)),
            pl.BlockSpec((BE, H), lambda i: (i, 0)),
            pl.BlockSpec((64, FD), lambda i: (0, 0)),
            pl.BlockSpec((1, FD), lambda i: (0, 0)),
            pl.BlockSpec((FD, FD), lambda i: (0, 0)),
            pl.BlockSpec((1, FD), lambda i: (0, 0)),
            pl.BlockSpec((FD, H * H), lambda i: (0, 0)),
            pl.BlockSpec((1, H * H), lambda i: (0, 0)),
            pl.BlockSpec((H * H, H), lambda i: (0, 0)),
        ],
        out_specs=pl.BlockSpec((BE, H), lambda i: (i, 0)),
        out_shape=jax.ShapeDtypeStruct((E_PAD, H), jnp.float32),
    )(ea_pad, xj, w1p, b1r, w2, b2r, w3, b3r, smat)


def _ffn_body(agg_ref, x_ref, wa1_ref, ba1_ref, wa2_ref, ba2_ref, out_ref):
    o = agg_ref[...]
    h2 = jax.nn.silu(jnp.dot(o, wa1_ref[...], preferred_element_type=jnp.float32)
                     + ba1_ref[...])
    h2 = jnp.dot(h2, wa2_ref[...], preferred_element_type=jnp.float32) + ba2_ref[...]
    out_ref[...] = h2 + x_ref[...]


def _ffn_pallas(agg, x, wa1, ba1r, wa2, ba2r):
    grid = N // N_BLK
    return pl.pallas_call(
        _ffn_body,
        grid=grid,
        in_specs=[
            pl.BlockSpec((N_BLK, H), lambda i: (i, 0)),
            pl.BlockSpec((N_BLK, H), lambda i: (i, 0)),
            pl.BlockSpec((H, H), lambda i: (0, 0)),
            pl.BlockSpec((1, H), lambda i: (0, 0)),
            pl.BlockSpec((H, H), lambda i: (0, 0)),
            pl.BlockSpec((1, H), lambda i: (0, 0)),
        ],
        out_specs=pl.BlockSpec((N_BLK, H), lambda i: (i, 0)),
        out_shape=jax.ShapeDtypeStruct((N, H), jnp.float32),
    )(agg, x, wa1, ba1r, wa2, ba2r)


def kernel(x, edge_index, edge_attr, W1, b1, W2, b2, W3, b3, Wa1, ba1, Wa2, ba2):
    src = edge_index[0].astype(jnp.int32)
    dst = edge_index[1].astype(jnp.int32)
    pad = E_PAD - E
    src_pad = jnp.concatenate([src, jnp.full((pad,), N, jnp.int32)])
    dst_pad = jnp.concatenate([dst, jnp.zeros((pad,), jnp.int32)])
    ea_pad = jnp.concatenate([edge_attr, jnp.zeros((pad, 1), jnp.float32)])
    x_aug = jnp.concatenate([x, jnp.zeros((1, H), jnp.float32)])

    w1p = jnp.zeros((64, FD), jnp.float32).at[:G].set(W1)
    smat = (jnp.arange(H * H)[:, None] // H == jnp.arange(H)[None, :]
            ).astype(jnp.float32)

    xj = x_aug[src_pad]
    msg = _msg_pallas(ea_pad, xj, w1p, b1.reshape(1, FD), W2,
                      b2.reshape(1, FD), W3, b3.reshape(1, H * H), smat)
    agg = jnp.zeros((N, H), jnp.float32).at[dst_pad].add(msg)
    return _ffn_pallas(agg, x, Wa1, ba1.reshape(1, H), Wa2, ba2.reshape(1, H))


# speedup vs baseline: 1.8830x; 1.8830x over previous
import functools

import jax
import jax.numpy as jnp
from jax import lax
from jax.experimental import pallas as pl
from jax.experimental.pallas import tpu as pltpu

N = 10000
E = 50000
H = 64
G = 50
FD = 128
START = 0.0
STOP = 5.0

NW = 32
CHUNK = 128
NCHUNK = 13
E_PAD = NW * NCHUNK * CHUNK
BE = 256
N_BLK = 1000

_COEFF = -0.5 / ((STOP - START) / (G - 1)) ** 2
_OFF_SCALE = (STOP - START) / (G - 1)
_LOG2 = 0.6931471805599453


def _ssp(v):
    return jnp.maximum(v, 0.0) + jnp.log1p(jnp.exp(-jnp.abs(v))) - _LOG2


def _msg_body(ea_ref, xj_ref, w1_ref, b1_ref, w2_ref, b2_ref, w3_ref, b3_ref,
              s_ref, msg_ref):
    ea = ea_ref[...]
    off = lax.broadcasted_iota(jnp.int32, (BE, 64), 1).astype(jnp.float32) * _OFF_SCALE
    d = ea - off
    rbf = jnp.exp(_COEFF * d * d)
    h = _ssp(jnp.dot(rbf, w1_ref[...], preferred_element_type=jnp.float32)
             + b1_ref[...])
    h = _ssp(jnp.dot(h, w2_ref[...], preferred_element_type=jnp.float32)
             + b2_ref[...])
    f3 = jnp.dot(h, w3_ref[...], preferred_element_type=jnp.float32) + b3_ref[...]
    xt = jnp.tile(xj_ref[...], (1, H))
    prod = f3 * xt
    msg_ref[...] = jnp.dot(prod, s_ref[...], preferred_element_type=jnp.float32)


def _msg_pallas(ea_pad, xj, w1p, b1r, w2, b2r, w3, b3r, smat):
    grid = E_PAD // BE
    return pl.pallas_call(
        _msg_body,
        grid=grid,
        in_specs=[
            pl.BlockSpec((BE, 1), lambda i: (i, 0)),
            pl.BlockSpec((BE, H), lambda i: (i, 0)),
            pl.BlockSpec((64, FD), lambda i: (0, 0)),
            pl.BlockSpec((1, FD), lambda i: (0, 0)),
            pl.BlockSpec((FD, FD), lambda i: (0, 0)),
            pl.BlockSpec((1, FD), lambda i: (0, 0)),
            pl.BlockSpec((FD, H * H), lambda i: (0, 0)),
            pl.BlockSpec((1, H * H), lambda i: (0, 0)),
            pl.BlockSpec((H * H, H), lambda i: (0, 0)),
        ],
        out_specs=pl.BlockSpec((BE, H), lambda i: (i, 0)),
        out_shape=jax.ShapeDtypeStruct((E_PAD, H), jnp.float32),
    )(ea_pad, xj, w1p, b1r, w2, b2r, w3, b3r, smat)


def _ffn_body(agg_ref, x_ref, wa1_ref, ba1_ref, wa2_ref, ba2_ref, out_ref):
    o = agg_ref[...]
    h2 = jax.nn.silu(jnp.dot(o, wa1_ref[...], preferred_element_type=jnp.float32)
                     + ba1_ref[...])
    h2 = jnp.dot(h2, wa2_ref[...], preferred_element_type=jnp.float32) + ba2_ref[...]
    out_ref[...] = h2 + x_ref[...]


def _ffn_pallas(agg, x, wa1, ba1r, wa2, ba2r):
    grid = N // N_BLK
    return pl.pallas_call(
        _ffn_body,
        grid=grid,
        in_specs=[
            pl.BlockSpec((N_BLK, H), lambda i: (i, 0)),
            pl.BlockSpec((N_BLK, H), lambda i: (i, 0)),
            pl.BlockSpec((H, H), lambda i: (0, 0)),
            pl.BlockSpec((1, H), lambda i: (0, 0)),
            pl.BlockSpec((H, H), lambda i: (0, 0)),
            pl.BlockSpec((1, H), lambda i: (0, 0)),
        ],
        out_specs=pl.BlockSpec((N_BLK, H), lambda i: (i, 0)),
        out_shape=jax.ShapeDtypeStruct((N, H), jnp.float32),
    )(agg, x, wa1, ba1r, wa2, ba2r)


def kernel(x, edge_index, edge_attr, W1, b1, W2, b2, W3, b3, Wa1, ba1, Wa2, ba2):
    src = edge_index[0].astype(jnp.int32)
    dst = edge_index[1].astype(jnp.int32)
    pad = E_PAD - E
    src_pad = jnp.concatenate([src, jnp.full((pad,), N, jnp.int32)])
    dst_pad = jnp.concatenate([dst, jnp.zeros((pad,), jnp.int32)])
    ea_pad = jnp.concatenate([edge_attr, jnp.zeros((pad, 1), jnp.float32)])
    x_aug = jnp.concatenate([x, jnp.zeros((1, H), jnp.float32)])

    w1p = jnp.zeros((64, FD), jnp.float32).at[:G].set(W1)
    smat = (jnp.arange(H * H)[:, None] // H == jnp.arange(H)[None, :]
            ).astype(jnp.float32)

    xj = x_aug[src_pad]
    msg = _msg_pallas(ea_pad, xj, w1p, b1.reshape(1, FD), W2,
                      b2.reshape(1, FD), W3, b3.reshape(1, H * H), smat)
    agg = jnp.zeros((N, H), jnp.float32).at[dst_pad].add(msg)
    return _ffn_pallas(agg, x, Wa1, ba1.reshape(1, H), Wa2, ba2.reshape(1, H))


# speedup vs baseline: 2.5067x; 1.3312x over previous
import functools

import jax
import jax.numpy as jnp
from jax import lax
from jax.experimental import pallas as pl
from jax.experimental.pallas import tpu as pltpu
from jax.experimental.pallas import tpu_sc as plsc

N = 10000
E = 50000
H = 64
G = 50
FD = 128
START = 0.0
STOP = 5.0

NW = 32
CHUNK = 128
NCHUNK = 13
E_PAD = NW * NCHUNK * CHUNK
BE = 256
N_BLK = 1000

_COEFF = -0.5 / ((STOP - START) / (G - 1)) ** 2
_OFF_SCALE = (STOP - START) / (G - 1)
_LOG2 = 0.6931471805599453


def _ssp(v):
    return jnp.maximum(v, 0.0) + jnp.log1p(jnp.exp(-jnp.abs(v))) - _LOG2


NC = 2
NS = 16
ROWS_PER_TILE = N // NS

_SC_MESH = dict(core_axis_name="c", subcore_axis_name="s")


def _sc_gather(x_aug, idx3):
    @functools.partial(
        pl.kernel,
        out_type=jax.ShapeDtypeStruct((E_PAD, H), jnp.float32),
        mesh=plsc.VectorSubcoreMesh(**_SC_MESH),
        scratch_types=[
            pltpu.VMEM((NCHUNK, CHUNK), jnp.int32),
            pltpu.VMEM((CHUNK, H), jnp.float32),
        ],
        compiler_params=pltpu.CompilerParams(use_tc_tiling_on_sc=False),
    )
    def k(x_hbm, idx_hbm, out_hbm, idx_v, rows_v):
        wid = lax.axis_index("s") * NC + lax.axis_index("c")
        pltpu.sync_copy(idx_hbm.at[wid], idx_v)
        for j in range(NCHUNK):
            pltpu.sync_copy(x_hbm.at[idx_v.at[j]], rows_v)
            pltpu.sync_copy(
                rows_v, out_hbm.at[pl.ds((wid * NCHUNK + j) * CHUNK, CHUNK)])

    return k(x_aug, idx3)


def _sc_scatter(msg, dst3, zeros_n):
    @functools.partial(
        pl.kernel,
        out_type=jax.ShapeDtypeStruct((NC, N, H), jnp.float32),
        mesh=plsc.VectorSubcoreMesh(**_SC_MESH),
        scratch_types=[
            pltpu.VMEM((NCHUNK, CHUNK), jnp.int32),
            pltpu.VMEM((CHUNK, H), jnp.float32),
            pltpu.VMEM_SHARED((N, H), jnp.float32),
        ],
        compiler_params=pltpu.CompilerParams(use_tc_tiling_on_sc=False),
    )
    def k(msg_hbm, idx_hbm, zeros_hbm, out_hbm, idx_v, msg_v, acc):
        cid = lax.axis_index("c")
        sid = lax.axis_index("s")
        wid = sid * NC + cid
        row0 = sid * ROWS_PER_TILE
        pltpu.sync_copy(zeros_hbm.at[pl.ds(row0, ROWS_PER_TILE)],
                        acc.at[pl.ds(row0, ROWS_PER_TILE)])
        pltpu.sync_copy(idx_hbm.at[wid], idx_v)
        plsc.subcore_barrier()
        for j in range(NCHUNK):
            pltpu.sync_copy(
                msg_hbm.at[pl.ds((wid * NCHUNK + j) * CHUNK, CHUNK)], msg_v)
            pltpu.sync_copy(msg_v, acc.at[idx_v.at[j]], add=True)
        plsc.subcore_barrier()
        pltpu.sync_copy(acc.at[pl.ds(row0, ROWS_PER_TILE)],
                        out_hbm.at[cid, pl.ds(row0, ROWS_PER_TILE)])

    return k(msg, dst3, zeros_n)


def _msg_body(ea_ref, xj_ref, w1_ref, b1_ref, w2_ref, b2_ref, w3_ref, b3_ref,
              s_ref, msg_ref):
    ea = ea_ref[...]
    off = lax.broadcasted_iota(jnp.int32, (BE, 64), 1).astype(jnp.float32) * _OFF_SCALE
    d = ea - off
    rbf = jnp.exp(_COEFF * d * d)
    h = _ssp(jnp.dot(rbf, w1_ref[...], preferred_element_type=jnp.float32)
             + b1_ref[...])
    h = _ssp(jnp.dot(h, w2_ref[...], preferred_element_type=jnp.float32)
             + b2_ref[...])
    f3 = jnp.dot(h, w3_ref[...], preferred_element_type=jnp.float32) + b3_ref[...]
    xt = jnp.tile(xj_ref[...], (1, H))
    prod = f3 * xt
    msg_ref[...] = jnp.dot(prod, s_ref[...], preferred_element_type=jnp.float32)


def _msg_pallas(ea_pad, xj, w1p, b1r, w2, b2r, w3, b3r, smat):
    grid = E_PAD // BE
    return pl.pallas_call(
        _msg_body,
        grid=grid,
        in_specs=[
            pl.BlockSpec((BE, 1), lambda i: (i, 0)),
            pl.BlockSpec((BE, H), lambda i: (i, 0)),
            pl.BlockSpec((64, FD), lambda i: (0, 0)),
            pl.BlockSpec((1, FD), lambda i: (0, 0)),
            pl.BlockSpec((FD, FD), lambda i: (0, 0)),
            pl.BlockSpec((1, FD), lambda i: (0, 0)),
            pl.BlockSpec((FD, H * H), lambda i: (0, 0)),
            pl.BlockSpec((1, H * H), lambda i: (0, 0)),
            pl.BlockSpec((H * H, H), lambda i: (0, 0)),
        ],
        out_specs=pl.BlockSpec((BE, H), lambda i: (i, 0)),
        out_shape=jax.ShapeDtypeStruct((E_PAD, H), jnp.float32),
    )(ea_pad, xj, w1p, b1r, w2, b2r, w3, b3r, smat)


def _ffn_body(p0_ref, p1_ref, x_ref, wa1_ref, ba1_ref, wa2_ref, ba2_ref, out_ref):
    o = p0_ref[...] + p1_ref[...]
    h2 = jax.nn.silu(jnp.dot(o, wa1_ref[...], preferred_element_type=jnp.float32)
                     + ba1_ref[...])
    h2 = jnp.dot(h2, wa2_ref[...], preferred_element_type=jnp.float32) + ba2_ref[...]
    out_ref[...] = h2 + x_ref[...]


def _ffn_pallas(p0, p1, x, wa1, ba1r, wa2, ba2r):
    grid = N // N_BLK
    return pl.pallas_call(
        _ffn_body,
        grid=grid,
        in_specs=[
            pl.BlockSpec((N_BLK, H), lambda i: (i, 0)),
            pl.BlockSpec((N_BLK, H), lambda i: (i, 0)),
            pl.BlockSpec((N_BLK, H), lambda i: (i, 0)),
            pl.BlockSpec((H, H), lambda i: (0, 0)),
            pl.BlockSpec((1, H), lambda i: (0, 0)),
            pl.BlockSpec((H, H), lambda i: (0, 0)),
            pl.BlockSpec((1, H), lambda i: (0, 0)),
        ],
        out_specs=pl.BlockSpec((N_BLK, H), lambda i: (i, 0)),
        out_shape=jax.ShapeDtypeStruct((N, H), jnp.float32),
    )(p0, p1, x, wa1, ba1r, wa2, ba2r)


def kernel(x, edge_index, edge_attr, W1, b1, W2, b2, W3, b3, Wa1, ba1, Wa2, ba2):
    src = edge_index[0].astype(jnp.int32)
    dst = edge_index[1].astype(jnp.int32)
    pad = E_PAD - E
    src_pad = jnp.concatenate([src, jnp.full((pad,), N, jnp.int32)])
    dst_pad = jnp.concatenate([dst, jnp.zeros((pad,), jnp.int32)])
    ea_pad = jnp.concatenate([edge_attr, jnp.zeros((pad, 1), jnp.float32)])
    x_aug = jnp.concatenate([x, jnp.zeros((1, H), jnp.float32)])

    w1p = jnp.zeros((64, FD), jnp.float32).at[:G].set(W1)
    smat = (jnp.arange(H * H)[:, None] // H == jnp.arange(H)[None, :]
            ).astype(jnp.float32)

    xj = _sc_gather(x_aug, src_pad.reshape(NW, NCHUNK, CHUNK))
    msg = _msg_pallas(ea_pad, xj, w1p, b1.reshape(1, FD), W2,
                      b2.reshape(1, FD), W3, b3.reshape(1, H * H), smat)
    partials = _sc_scatter(msg, dst_pad.reshape(NW, NCHUNK, CHUNK),
                           jnp.zeros((N, H), jnp.float32))
    return _ffn_pallas(partials[0], partials[1], x, Wa1,
                       ba1.reshape(1, H), Wa2, ba2.reshape(1, H))


# speedup vs baseline: 2.5282x; 1.0086x over previous
import functools

import jax
import jax.numpy as jnp
from jax import lax
from jax.experimental import pallas as pl
from jax.experimental.pallas import tpu as pltpu
from jax.experimental.pallas import tpu_sc as plsc

N = 10000
E = 50000
H = 64
G = 50
FD = 128
START = 0.0
STOP = 5.0

NW = 32
CHUNK = 128
NCHUNK = 13
E_PAD = NW * NCHUNK * CHUNK
BE = 256
N_BLK = 1000

_COEFF = -0.5 / ((STOP - START) / (G - 1)) ** 2
_OFF_SCALE = (STOP - START) / (G - 1)
_LOG2 = 0.6931471805599453


def _ssp(v):
    return jnp.maximum(v, 0.0) + jnp.log1p(jnp.exp(-jnp.abs(v))) - _LOG2


NC = 2
NS = 16
ROWS_PER_TILE = N // NS

_SC_MESH = dict(core_axis_name="c", subcore_axis_name="s")


def _sc_gather(x_aug, idx3):
    @functools.partial(
        pl.kernel,
        out_type=jax.ShapeDtypeStruct((E_PAD, H), jnp.float32),
        mesh=plsc.VectorSubcoreMesh(**_SC_MESH),
        scratch_types=[
            pltpu.VMEM((NCHUNK, CHUNK), jnp.int32),
            pltpu.VMEM((CHUNK, H), jnp.float32),
        ],
        compiler_params=pltpu.CompilerParams(use_tc_tiling_on_sc=False),
    )
    def k(x_hbm, idx_hbm, out_hbm, idx_v, rows_v):
        wid = lax.axis_index("s") * NC + lax.axis_index("c")
        pltpu.sync_copy(idx_hbm.at[wid], idx_v)
        for j in range(NCHUNK):
            pltpu.sync_copy(x_hbm.at[idx_v.at[j]], rows_v)
            pltpu.sync_copy(
                rows_v, out_hbm.at[pl.ds((wid * NCHUNK + j) * CHUNK, CHUNK)])

    return k(x_aug, idx3)


def _sc_scatter(msg, dst3, zeros_n):
    @functools.partial(
        pl.kernel,
        out_type=jax.ShapeDtypeStruct((NC, N, H), jnp.float32),
        mesh=plsc.VectorSubcoreMesh(**_SC_MESH),
        scratch_types=[
            pltpu.VMEM((NCHUNK, CHUNK), jnp.int32),
            pltpu.VMEM((CHUNK, H), jnp.float32),
            pltpu.VMEM_SHARED((N, H), jnp.float32),
        ],
        compiler_params=pltpu.CompilerParams(use_tc_tiling_on_sc=False),
    )
    def k(msg_hbm, idx_hbm, zeros_hbm, out_hbm, idx_v, msg_v, acc):
        cid = lax.axis_index("c")
        sid = lax.axis_index("s")
        wid = sid * NC + cid
        row0 = sid * ROWS_PER_TILE
        pltpu.sync_copy(zeros_hbm.at[pl.ds(row0, ROWS_PER_TILE)],
                        acc.at[pl.ds(row0, ROWS_PER_TILE)])
        pltpu.sync_copy(idx_hbm.at[wid], idx_v)
        plsc.subcore_barrier()
        for j in range(NCHUNK):
            pltpu.sync_copy(
                msg_hbm.at[pl.ds((wid * NCHUNK + j) * CHUNK, CHUNK)], msg_v)
            pltpu.sync_copy(msg_v, acc.at[idx_v.at[j]], add=True)
        plsc.subcore_barrier()
        pltpu.sync_copy(acc.at[pl.ds(row0, ROWS_PER_TILE)],
                        out_hbm.at[cid, pl.ds(row0, ROWS_PER_TILE)])

    return k(msg, dst3, zeros_n)


def _msg_body(ea_ref, xj_ref, w1_ref, b1_ref, w2_ref, b2_ref, w3_ref, b3_ref,
              s_ref, msg_ref):
    ea = ea_ref[...]
    off = lax.broadcasted_iota(jnp.int32, (BE, 64), 1).astype(jnp.float32) * _OFF_SCALE
    d = ea - off
    rbf = jnp.exp(_COEFF * d * d)
    h = _ssp(jnp.dot(rbf, w1_ref[...], preferred_element_type=jnp.float32)
             + b1_ref[...])
    h = _ssp(jnp.dot(h, w2_ref[...], preferred_element_type=jnp.float32)
             + b2_ref[...])
    f3 = jnp.dot(h.astype(jnp.bfloat16), w3_ref[...],
                 preferred_element_type=jnp.float32) + b3_ref[...]
    xt = jnp.tile(xj_ref[...], (1, H))
    prod = (f3 * xt).astype(jnp.bfloat16)
    msg_ref[...] = jnp.dot(prod, s_ref[...], preferred_element_type=jnp.float32)


def _msg_pallas(ea_pad, xj, w1p, b1r, w2, b2r, w3, b3r, smat):
    grid = E_PAD // BE
    return pl.pallas_call(
        _msg_body,
        grid=grid,
        in_specs=[
            pl.BlockSpec((BE, 1), lambda i: (i, 0)),
            pl.BlockSpec((BE, H), lambda i: (i, 0)),
            pl.BlockSpec((64, FD), lambda i: (0, 0)),
            pl.BlockSpec((1, FD), lambda i: (0, 0)),
            pl.BlockSpec((FD, FD), lambda i: (0, 0)),
            pl.BlockSpec((1, FD), lambda i: (0, 0)),
            pl.BlockSpec((FD, H * H), lambda i: (0, 0)),
            pl.BlockSpec((1, H * H), lambda i: (0, 0)),
            pl.BlockSpec((H * H, H), lambda i: (0, 0)),
        ],
        out_specs=pl.BlockSpec((BE, H), lambda i: (i, 0)),
        out_shape=jax.ShapeDtypeStruct((E_PAD, H), jnp.float32),
    )(ea_pad, xj, w1p, b1r, w2, b2r, w3, b3r, smat)


def _ffn_body(p0_ref, p1_ref, x_ref, wa1_ref, ba1_ref, wa2_ref, ba2_ref, out_ref):
    o = p0_ref[...] + p1_ref[...]
    h2 = jax.nn.silu(jnp.dot(o, wa1_ref[...], preferred_element_type=jnp.float32)
                     + ba1_ref[...])
    h2 = jnp.dot(h2, wa2_ref[...], preferred_element_type=jnp.float32) + ba2_ref[...]
    out_ref[...] = h2 + x_ref[...]


def _ffn_pallas(p0, p1, x, wa1, ba1r, wa2, ba2r):
    grid = N // N_BLK
    return pl.pallas_call(
        _ffn_body,
        grid=grid,
        in_specs=[
            pl.BlockSpec((N_BLK, H), lambda i: (i, 0)),
            pl.BlockSpec((N_BLK, H), lambda i: (i, 0)),
            pl.BlockSpec((N_BLK, H), lambda i: (i, 0)),
            pl.BlockSpec((H, H), lambda i: (0, 0)),
            pl.BlockSpec((1, H), lambda i: (0, 0)),
            pl.BlockSpec((H, H), lambda i: (0, 0)),
            pl.BlockSpec((1, H), lambda i: (0, 0)),
        ],
        out_specs=pl.BlockSpec((N_BLK, H), lambda i: (i, 0)),
        out_shape=jax.ShapeDtypeStruct((N, H), jnp.float32),
    )(p0, p1, x, wa1, ba1r, wa2, ba2r)


def kernel(x, edge_index, edge_attr, W1, b1, W2, b2, W3, b3, Wa1, ba1, Wa2, ba2):
    src = edge_index[0].astype(jnp.int32)
    dst = edge_index[1].astype(jnp.int32)
    pad = E_PAD - E
    src_pad = jnp.concatenate([src, jnp.full((pad,), N, jnp.int32)])
    dst_pad = jnp.concatenate([dst, jnp.zeros((pad,), jnp.int32)])
    ea_pad = jnp.concatenate([edge_attr, jnp.zeros((pad, 1), jnp.float32)])
    x_aug = jnp.concatenate([x, jnp.zeros((1, H), jnp.float32)])

    w1p = jnp.zeros((64, FD), jnp.float32).at[:G].set(W1)
    smat = (jnp.arange(H * H)[:, None] // H == jnp.arange(H)[None, :]
            ).astype(jnp.bfloat16)

    xj = _sc_gather(x_aug, src_pad.reshape(NW, NCHUNK, CHUNK))
    msg = _msg_pallas(ea_pad, xj, w1p, b1.reshape(1, FD), W2,
                      b2.reshape(1, FD), W3.astype(jnp.bfloat16),
                      b3.reshape(1, H * H), smat)
    partials = _sc_scatter(msg, dst_pad.reshape(NW, NCHUNK, CHUNK),
                           jnp.zeros((N, H), jnp.float32))
    return _ffn_pallas(partials[0], partials[1], x, Wa1,
                       ba1.reshape(1, H), Wa2, ba2.reshape(1, H))


# speedup vs baseline: 2.7384x; 1.0831x over previous
import functools

import jax
import jax.numpy as jnp
from jax import lax
from jax.experimental import pallas as pl
from jax.experimental.pallas import tpu as pltpu
from jax.experimental.pallas import tpu_sc as plsc

N = 10000
E = 50000
H = 64
G = 50
FD = 128
START = 0.0
STOP = 5.0

NW = 32
CHUNK = 128
NCHUNK = 13
E_PAD = NW * NCHUNK * CHUNK
BE = 512
N_BLK = 1000

_COEFF = -0.5 / ((STOP - START) / (G - 1)) ** 2
_OFF_SCALE = (STOP - START) / (G - 1)
_LOG2 = 0.6931471805599453


def _ssp(v):
    return jnp.maximum(v, 0.0) + jnp.log1p(jnp.exp(-jnp.abs(v))) - _LOG2


NC = 2
NS = 16
ROWS_PER_TILE = N // NS

_SC_MESH = dict(core_axis_name="c", subcore_axis_name="s")


def _sc_gather(x_aug, idx3):
    @functools.partial(
        pl.kernel,
        out_type=jax.ShapeDtypeStruct((E_PAD, H), jnp.float32),
        mesh=plsc.VectorSubcoreMesh(**_SC_MESH),
        scratch_types=[
            pltpu.VMEM((NCHUNK, CHUNK), jnp.int32),
            pltpu.VMEM((CHUNK, H), jnp.float32),
        ],
        compiler_params=pltpu.CompilerParams(use_tc_tiling_on_sc=False),
    )
    def k(x_hbm, idx_hbm, out_hbm, idx_v, rows_v):
        wid = lax.axis_index("s") * NC + lax.axis_index("c")
        pltpu.sync_copy(idx_hbm.at[wid], idx_v)
        for j in range(NCHUNK):
            pltpu.sync_copy(x_hbm.at[idx_v.at[j]], rows_v)
            pltpu.sync_copy(
                rows_v, out_hbm.at[pl.ds((wid * NCHUNK + j) * CHUNK, CHUNK)])

    return k(x_aug, idx3)


def _sc_scatter(msg, dst3, zeros_n):
    @functools.partial(
        pl.kernel,
        out_type=jax.ShapeDtypeStruct((NC, N, H), jnp.float32),
        mesh=plsc.VectorSubcoreMesh(**_SC_MESH),
        scratch_types=[
            pltpu.VMEM((NCHUNK, CHUNK), jnp.int32),
            pltpu.VMEM((CHUNK, H), jnp.float32),
            pltpu.VMEM_SHARED((N, H), jnp.float32),
        ],
        compiler_params=pltpu.CompilerParams(use_tc_tiling_on_sc=False),
    )
    def k(msg_hbm, idx_hbm, zeros_hbm, out_hbm, idx_v, msg_v, acc):
        cid = lax.axis_index("c")
        sid = lax.axis_index("s")
        wid = sid * NC + cid
        row0 = sid * ROWS_PER_TILE
        pltpu.sync_copy(zeros_hbm.at[pl.ds(row0, ROWS_PER_TILE)],
                        acc.at[pl.ds(row0, ROWS_PER_TILE)])
        pltpu.sync_copy(idx_hbm.at[wid], idx_v)
        plsc.subcore_barrier()
        for j in range(NCHUNK):
            pltpu.sync_copy(
                msg_hbm.at[pl.ds((wid * NCHUNK + j) * CHUNK, CHUNK)], msg_v)
            pltpu.sync_copy(msg_v, acc.at[idx_v.at[j]], add=True)
        plsc.subcore_barrier()
        pltpu.sync_copy(acc.at[pl.ds(row0, ROWS_PER_TILE)],
                        out_hbm.at[cid, pl.ds(row0, ROWS_PER_TILE)])

    return k(msg, dst3, zeros_n)


def _msg_body(ea_ref, xj_ref, w1_ref, b1_ref, w2_ref, b2_ref, w3_ref, b3t_ref,
              s_ref, msg_ref):
    ea = ea_ref[...]
    off = lax.broadcasted_iota(jnp.int32, (BE, 64), 1).astype(jnp.float32) * _OFF_SCALE
    d = ea - off
    rbf = jnp.exp(_COEFF * d * d)
    h = _ssp(jnp.dot(rbf, w1_ref[...], preferred_element_type=jnp.float32)
             + b1_ref[...])
    h = _ssp(jnp.dot(h, w2_ref[...], preferred_element_type=jnp.float32)
             + b2_ref[...])
    f3 = jnp.dot(h.astype(jnp.bfloat16), w3_ref[...],
                 preferred_element_type=jnp.float32).astype(jnp.bfloat16)
    xj = xj_ref[...]
    xt = jnp.tile(xj.astype(jnp.bfloat16), (1, H))
    prod = f3 * xt
    msg_ref[...] = (
        jnp.dot(prod, s_ref[...], preferred_element_type=jnp.float32)
        + jnp.dot(xj, b3t_ref[...], preferred_element_type=jnp.float32))


def _msg_pallas(ea_pad, xj, w1p, b1r, w2, b2r, w3, b3r, smat):
    grid = E_PAD // BE
    return pl.pallas_call(
        _msg_body,
        grid=grid,
        in_specs=[
            pl.BlockSpec((BE, 1), lambda i: (i, 0)),
            pl.BlockSpec((BE, H), lambda i: (i, 0)),
            pl.BlockSpec((64, FD), lambda i: (0, 0)),
            pl.BlockSpec((1, FD), lambda i: (0, 0)),
            pl.BlockSpec((FD, FD), lambda i: (0, 0)),
            pl.BlockSpec((1, FD), lambda i: (0, 0)),
            pl.BlockSpec((FD, H * H), lambda i: (0, 0)),
            pl.BlockSpec((H, H), lambda i: (0, 0)),
            pl.BlockSpec((H * H, H), lambda i: (0, 0)),
        ],
        out_specs=pl.BlockSpec((BE, H), lambda i: (i, 0)),
        out_shape=jax.ShapeDtypeStruct((E_PAD, H), jnp.float32),
    )(ea_pad, xj, w1p, b1r, w2, b2r, w3, b3r, smat)


def _ffn_body(p0_ref, p1_ref, x_ref, wa1_ref, ba1_ref, wa2_ref, ba2_ref, out_ref):
    o = p0_ref[...] + p1_ref[...]
    h2 = jax.nn.silu(jnp.dot(o, wa1_ref[...], preferred_element_type=jnp.float32)
                     + ba1_ref[...])
    h2 = jnp.dot(h2, wa2_ref[...], preferred_element_type=jnp.float32) + ba2_ref[...]
    out_ref[...] = h2 + x_ref[...]


def _ffn_pallas(p0, p1, x, wa1, ba1r, wa2, ba2r):
    grid = N // N_BLK
    return pl.pallas_call(
        _ffn_body,
        grid=grid,
        in_specs=[
            pl.BlockSpec((N_BLK, H), lambda i: (i, 0)),
            pl.BlockSpec((N_BLK, H), lambda i: (i, 0)),
            pl.BlockSpec((N_BLK, H), lambda i: (i, 0)),
            pl.BlockSpec((H, H), lambda i: (0, 0)),
            pl.BlockSpec((1, H), lambda i: (0, 0)),
            pl.BlockSpec((H, H), lambda i: (0, 0)),
            pl.BlockSpec((1, H), lambda i: (0, 0)),
        ],
        out_specs=pl.BlockSpec((N_BLK, H), lambda i: (i, 0)),
        out_shape=jax.ShapeDtypeStruct((N, H), jnp.float32),
    )(p0, p1, x, wa1, ba1r, wa2, ba2r)


def kernel(x, edge_index, edge_attr, W1, b1, W2, b2, W3, b3, Wa1, ba1, Wa2, ba2):
    src = edge_index[0].astype(jnp.int32)
    dst = edge_index[1].astype(jnp.int32)
    pad = E_PAD - E
    src_pad = jnp.concatenate([src, jnp.full((pad,), N, jnp.int32)])
    dst_pad = jnp.concatenate([dst, jnp.zeros((pad,), jnp.int32)])
    ea_pad = jnp.concatenate([edge_attr, jnp.zeros((pad, 1), jnp.float32)])
    x_aug = jnp.concatenate([x, jnp.zeros((1, H), jnp.float32)])

    w1p = jnp.zeros((64, FD), jnp.float32).at[:G].set(W1)
    smat = (jnp.arange(H * H)[:, None] // H == jnp.arange(H)[None, :]
            ).astype(jnp.bfloat16)

    xj = _sc_gather(x_aug, src_pad.reshape(NW, NCHUNK, CHUNK))
    msg = _msg_pallas(ea_pad, xj, w1p, b1.reshape(1, FD), W2,
                      b2.reshape(1, FD), W3.astype(jnp.bfloat16),
                      b3.reshape(H, H).T, smat)
    partials = _sc_scatter(msg, dst_pad.reshape(NW, NCHUNK, CHUNK),
                           jnp.zeros((N, H), jnp.float32))
    return _ffn_pallas(partials[0], partials[1], x, Wa1,
                       ba1.reshape(1, H), Wa2, ba2.reshape(1, H))


# speedup vs baseline: 2.7633x; 1.0091x over previous
import functools

import jax
import jax.numpy as jnp
from jax import lax
from jax.experimental import pallas as pl
from jax.experimental.pallas import tpu as pltpu
from jax.experimental.pallas import tpu_sc as plsc

N = 10000
E = 50000
H = 64
G = 50
FD = 128
START = 0.0
STOP = 5.0

NW = 32
CHUNK = 128
NCHUNK = 13
E_PAD = NW * NCHUNK * CHUNK
BE = 512
N_BLK = 1000

_COEFF = -0.5 / ((STOP - START) / (G - 1)) ** 2
_OFF_SCALE = (STOP - START) / (G - 1)
_LOG2 = 0.6931471805599453


def _ssp(v):
    return jnp.maximum(v, 0.0) + jnp.log1p(jnp.exp(-jnp.abs(v))) - _LOG2


NC = 2
NS = 16
ROWS_PER_TILE = N // NS

_SC_MESH = dict(core_axis_name="c", subcore_axis_name="s")


EPW = NCHUNK * CHUNK


def _sc_gather(x_aug, idx3):
    @functools.partial(
        pl.kernel,
        out_type=jax.ShapeDtypeStruct((E_PAD, H), jnp.float32),
        mesh=plsc.VectorSubcoreMesh(**_SC_MESH),
        scratch_types=[
            pltpu.VMEM((NCHUNK, CHUNK), jnp.int32),
            pltpu.VMEM((EPW, H), jnp.float32),
            pltpu.SemaphoreType.DMA,
        ],
        compiler_params=pltpu.CompilerParams(use_tc_tiling_on_sc=False),
    )
    def k(x_hbm, idx_hbm, out_hbm, idx_v, rows_v, sem):
        wid = lax.axis_index("s") * NC + lax.axis_index("c")
        pltpu.sync_copy(idx_hbm.at[wid], idx_v)
        descs = [
            pltpu.async_copy(x_hbm.at[idx_v.at[j]],
                             rows_v.at[pl.ds(j * CHUNK, CHUNK)], sem)
            for j in range(NCHUNK)
        ]
        for d in descs:
            d.wait()
        pltpu.sync_copy(rows_v, out_hbm.at[pl.ds(wid * EPW, EPW)])

    return k(x_aug, idx3)


def _sc_scatter(msg, dst3, zeros_n):
    @functools.partial(
        pl.kernel,
        out_type=jax.ShapeDtypeStruct((NC, N, H), jnp.float32),
        mesh=plsc.VectorSubcoreMesh(**_SC_MESH),
        scratch_types=[
            pltpu.VMEM((NCHUNK, CHUNK), jnp.int32),
            pltpu.VMEM((2, CHUNK, H), jnp.float32),
            pltpu.VMEM_SHARED((N, H), jnp.float32),
            pltpu.SemaphoreType.DMA,
        ],
        compiler_params=pltpu.CompilerParams(use_tc_tiling_on_sc=False),
    )
    def k(msg_hbm, idx_hbm, zeros_hbm, out_hbm, idx_v, msg_v, acc, sem):
        cid = lax.axis_index("c")
        sid = lax.axis_index("s")
        wid = sid * NC + cid
        row0 = sid * ROWS_PER_TILE
        pltpu.sync_copy(zeros_hbm.at[pl.ds(0, ROWS_PER_TILE)],
                        acc.at[pl.ds(row0, ROWS_PER_TILE)])
        pltpu.sync_copy(idx_hbm.at[wid], idx_v)
        plsc.subcore_barrier()
        load = pltpu.async_copy(msg_hbm.at[wid, 0], msg_v.at[0], sem)
        for j in range(NCHUNK):
            load.wait()
            if j + 1 < NCHUNK:
                load = pltpu.async_copy(msg_hbm.at[wid, j + 1],
                                        msg_v.at[(j + 1) % 2], sem)
            pltpu.sync_copy(msg_v.at[j % 2], acc.at[idx_v.at[j]], add=True)
        plsc.subcore_barrier()
        pltpu.sync_copy(acc.at[pl.ds(row0, ROWS_PER_TILE)],
                        out_hbm.at[cid, pl.ds(row0, ROWS_PER_TILE)])

    return k(msg, dst3, zeros_n)


def _msg_body(ea_ref, xj_ref, w1_ref, b1_ref, w2_ref, b2_ref, w3_ref, b3t_ref,
              s_ref, msg_ref):
    ea = ea_ref[...]
    off = lax.broadcasted_iota(jnp.int32, (BE, 64), 1).astype(jnp.float32) * _OFF_SCALE
    d = ea - off
    rbf = jnp.exp(_COEFF * d * d)
    h = _ssp(jnp.dot(rbf, w1_ref[...], preferred_element_type=jnp.float32)
             + b1_ref[...])
    h = _ssp(jnp.dot(h, w2_ref[...], preferred_element_type=jnp.float32)
             + b2_ref[...])
    f3 = jnp.dot(h.astype(jnp.bfloat16), w3_ref[...],
                 preferred_element_type=jnp.float32).astype(jnp.bfloat16)
    xj = xj_ref[...]
    xt = jnp.tile(xj.astype(jnp.bfloat16), (1, H))
    prod = f3 * xt
    msg_ref[...] = (
        jnp.dot(prod, s_ref[...], preferred_element_type=jnp.float32)
        + jnp.dot(xj, b3t_ref[...], preferred_element_type=jnp.float32))


def _msg_pallas(ea_pad, xj, w1p, b1r, w2, b2r, w3, b3r, smat):
    grid = E_PAD // BE
    return pl.pallas_call(
        _msg_body,
        grid=grid,
        in_specs=[
            pl.BlockSpec((BE, 1), lambda i: (i, 0)),
            pl.BlockSpec((BE, H), lambda i: (i, 0)),
            pl.BlockSpec((64, FD), lambda i: (0, 0)),
            pl.BlockSpec((1, FD), lambda i: (0, 0)),
            pl.BlockSpec((FD, FD), lambda i: (0, 0)),
            pl.BlockSpec((1, FD), lambda i: (0, 0)),
            pl.BlockSpec((FD, H * H), lambda i: (0, 0)),
            pl.BlockSpec((H, H), lambda i: (0, 0)),
            pl.BlockSpec((H * H, H), lambda i: (0, 0)),
        ],
        out_specs=pl.BlockSpec((BE, H), lambda i: (i, 0)),
        out_shape=jax.ShapeDtypeStruct((E_PAD, H), jnp.float32),
    )(ea_pad, xj, w1p, b1r, w2, b2r, w3, b3r, smat)


def _ffn_body(p0_ref, p1_ref, x_ref, wa1_ref, ba1_ref, wa2_ref, ba2_ref, out_ref):
    o = p0_ref[...] + p1_ref[...]
    h2 = jax.nn.silu(jnp.dot(o, wa1_ref[...], preferred_element_type=jnp.float32)
                     + ba1_ref[...])
    h2 = jnp.dot(h2, wa2_ref[...], preferred_element_type=jnp.float32) + ba2_ref[...]
    out_ref[...] = h2 + x_ref[...]


def _ffn_pallas(p0, p1, x, wa1, ba1r, wa2, ba2r):
    grid = N // N_BLK
    return pl.pallas_call(
        _ffn_body,
        grid=grid,
        in_specs=[
            pl.BlockSpec((N_BLK, H), lambda i: (i, 0)),
            pl.BlockSpec((N_BLK, H), lambda i: (i, 0)),
            pl.BlockSpec((N_BLK, H), lambda i: (i, 0)),
            pl.BlockSpec((H, H), lambda i: (0, 0)),
            pl.BlockSpec((1, H), lambda i: (0, 0)),
            pl.BlockSpec((H, H), lambda i: (0, 0)),
            pl.BlockSpec((1, H), lambda i: (0, 0)),
        ],
        out_specs=pl.BlockSpec((N_BLK, H), lambda i: (i, 0)),
        out_shape=jax.ShapeDtypeStruct((N, H), jnp.float32),
    )(p0, p1, x, wa1, ba1r, wa2, ba2r)


def kernel(x, edge_index, edge_attr, W1, b1, W2, b2, W3, b3, Wa1, ba1, Wa2, ba2):
    src = edge_index[0].astype(jnp.int32)
    dst = edge_index[1].astype(jnp.int32)
    pad = E_PAD - E
    src_pad = jnp.concatenate([src, jnp.full((pad,), N, jnp.int32)])
    dst_pad = jnp.concatenate([dst, jnp.zeros((pad,), jnp.int32)])
    ea_pad = jnp.concatenate([edge_attr, jnp.zeros((pad, 1), jnp.float32)])
    x_aug = jnp.concatenate([x, jnp.zeros((1, H), jnp.float32)])

    w1p = jnp.zeros((64, FD), jnp.float32).at[:G].set(W1)
    smat = (jnp.arange(H * H)[:, None] // H == jnp.arange(H)[None, :]
            ).astype(jnp.bfloat16)

    xj = _sc_gather(x_aug, src_pad.reshape(NW, NCHUNK, CHUNK))
    msg = _msg_pallas(ea_pad, xj, w1p, b1.reshape(1, FD), W2,
                      b2.reshape(1, FD), W3.astype(jnp.bfloat16),
                      b3.reshape(H, H).T, smat)
    partials = _sc_scatter(msg.reshape(NW, NCHUNK, CHUNK, H),
                           dst_pad.reshape(NW, NCHUNK, CHUNK),
                           jnp.zeros((ROWS_PER_TILE, H), jnp.float32))
    return _ffn_pallas(partials[0], partials[1], x, Wa1,
                       ba1.reshape(1, H), Wa2, ba2.reshape(1, H))


# speedup vs baseline: 2.8378x; 1.0270x over previous
import functools

import jax
import jax.numpy as jnp
from jax import lax
from jax.experimental import pallas as pl
from jax.experimental.pallas import tpu as pltpu
from jax.experimental.pallas import tpu_sc as plsc

N = 10000
E = 50000
H = 64
G = 50
FD = 128
START = 0.0
STOP = 5.0

NW = 32
CHUNK = 128
NCHUNK = 13
E_PAD = NW * NCHUNK * CHUNK
BE = 512
N_BLK = 1000

_COEFF = -0.5 / ((STOP - START) / (G - 1)) ** 2
_OFF_SCALE = (STOP - START) / (G - 1)
_LOG2 = 0.6931471805599453


def _ssp(v):
    return jnp.maximum(v, 0.0) + jnp.log1p(jnp.exp(-jnp.abs(v))) - _LOG2


NC = 2
NS = 16
N_ACC = N + 16
ROWS_PER_TILE = N_ACC // NS

_SC_MESH = dict(core_axis_name="c", subcore_axis_name="s")


EPW = NCHUNK * CHUNK


def _sc_gather(x_aug, idx3):
    @functools.partial(
        pl.kernel,
        out_type=jax.ShapeDtypeStruct((E_PAD, H), jnp.bfloat16),
        mesh=plsc.VectorSubcoreMesh(**_SC_MESH),
        scratch_types=[
            pltpu.VMEM((NCHUNK, CHUNK), jnp.int32),
            pltpu.VMEM((EPW, H), jnp.bfloat16),
            pltpu.SemaphoreType.DMA,
        ],
        compiler_params=pltpu.CompilerParams(use_tc_tiling_on_sc=False),
    )
    def k(x_hbm, idx_hbm, out_hbm, idx_v, rows_v, sem):
        wid = lax.axis_index("s") * NC + lax.axis_index("c")
        pltpu.sync_copy(idx_hbm.at[wid], idx_v)
        descs = [
            pltpu.async_copy(x_hbm.at[idx_v.at[j]],
                             rows_v.at[pl.ds(j * CHUNK, CHUNK)], sem)
            for j in range(NCHUNK)
        ]
        for d in descs:
            d.wait()
        pltpu.sync_copy(rows_v, out_hbm.at[pl.ds(wid * EPW, EPW)])

    return k(x_aug, idx3)


def _sc_scatter(msg, dst3, zeros_n):
    @functools.partial(
        pl.kernel,
        out_type=jax.ShapeDtypeStruct((NC, N_ACC, H), jnp.bfloat16),
        mesh=plsc.VectorSubcoreMesh(**_SC_MESH),
        scratch_types=[
            pltpu.VMEM((NCHUNK, CHUNK), jnp.int32),
            pltpu.VMEM((2, CHUNK, H), jnp.bfloat16),
            pltpu.VMEM_SHARED((N_ACC, H), jnp.bfloat16),
            pltpu.SemaphoreType.DMA,
        ],
        compiler_params=pltpu.CompilerParams(use_tc_tiling_on_sc=False),
    )
    def k(msg_hbm, idx_hbm, zeros_hbm, out_hbm, idx_v, msg_v, acc, sem):
        cid = lax.axis_index("c")
        sid = lax.axis_index("s")
        wid = sid * NC + cid
        row0 = sid * ROWS_PER_TILE
        pltpu.sync_copy(zeros_hbm.at[pl.ds(0, ROWS_PER_TILE)],
                        acc.at[pl.ds(row0, ROWS_PER_TILE)])
        pltpu.sync_copy(idx_hbm.at[wid], idx_v)
        plsc.subcore_barrier()
        load = pltpu.async_copy(msg_hbm.at[wid, 0], msg_v.at[0], sem)
        for j in range(NCHUNK):
            load.wait()
            if j + 1 < NCHUNK:
                load = pltpu.async_copy(msg_hbm.at[wid, j + 1],
                                        msg_v.at[(j + 1) % 2], sem)
            pltpu.sync_copy(msg_v.at[j % 2], acc.at[idx_v.at[j]], add=True)
        plsc.subcore_barrier()
        pltpu.sync_copy(acc.at[pl.ds(row0, ROWS_PER_TILE)],
                        out_hbm.at[cid, pl.ds(row0, ROWS_PER_TILE)])

    return k(msg, dst3, zeros_n)


def _msg_body(ea_ref, xj_ref, w1_ref, b1_ref, w2_ref, b2_ref, w3_ref, b3t_ref,
              s_ref, msg_ref):
    ea = ea_ref[...]
    off = lax.broadcasted_iota(jnp.int32, (BE, 64), 1).astype(jnp.float32) * _OFF_SCALE
    d = ea - off
    rbf = jnp.exp(_COEFF * d * d)
    h = _ssp(jnp.dot(rbf, w1_ref[...], preferred_element_type=jnp.float32)
             + b1_ref[...])
    h = _ssp(jnp.dot(h, w2_ref[...], preferred_element_type=jnp.float32)
             + b2_ref[...])
    f3 = jnp.dot(h.astype(jnp.bfloat16), w3_ref[...],
                 preferred_element_type=jnp.float32).astype(jnp.bfloat16)
    xj = xj_ref[...]
    xt = jnp.tile(xj, (1, H))
    prod = f3 * xt
    msg_ref[...] = (
        jnp.dot(prod, s_ref[...], preferred_element_type=jnp.float32)
        + jnp.dot(xj, b3t_ref[...], preferred_element_type=jnp.float32)
    ).astype(jnp.bfloat16)


def _msg_pallas(ea_pad, xj, w1p, b1r, w2, b2r, w3, b3r, smat):
    grid = E_PAD // BE
    return pl.pallas_call(
        _msg_body,
        grid=grid,
        in_specs=[
            pl.BlockSpec((BE, 1), lambda i: (i, 0)),
            pl.BlockSpec((BE, H), lambda i: (i, 0)),
            pl.BlockSpec((64, FD), lambda i: (0, 0)),
            pl.BlockSpec((1, FD), lambda i: (0, 0)),
            pl.BlockSpec((FD, FD), lambda i: (0, 0)),
            pl.BlockSpec((1, FD), lambda i: (0, 0)),
            pl.BlockSpec((FD, H * H), lambda i: (0, 0)),
            pl.BlockSpec((H, H), lambda i: (0, 0)),
            pl.BlockSpec((H * H, H), lambda i: (0, 0)),
        ],
        out_specs=pl.BlockSpec((BE, H), lambda i: (i, 0)),
        out_shape=jax.ShapeDtypeStruct((E_PAD, H), jnp.bfloat16),
    )(ea_pad, xj, w1p, b1r, w2, b2r, w3, b3r, smat)


def _ffn_body(p_ref, x_ref, wa1_ref, ba1_ref, wa2_ref, ba2_ref, out_ref):
    o = p_ref[0].astype(jnp.float32) + p_ref[1].astype(jnp.float32)
    h2 = jax.nn.silu(jnp.dot(o, wa1_ref[...], preferred_element_type=jnp.float32)
                     + ba1_ref[...])
    h2 = jnp.dot(h2, wa2_ref[...], preferred_element_type=jnp.float32) + ba2_ref[...]
    out_ref[...] = h2 + x_ref[...]


def _ffn_pallas(p, x, wa1, ba1r, wa2, ba2r):
    grid = N // N_BLK
    return pl.pallas_call(
        _ffn_body,
        grid=grid,
        in_specs=[
            pl.BlockSpec((NC, N_BLK, H), lambda i: (0, i, 0)),
            pl.BlockSpec((N_BLK, H), lambda i: (i, 0)),
            pl.BlockSpec((H, H), lambda i: (0, 0)),
            pl.BlockSpec((1, H), lambda i: (0, 0)),
            pl.BlockSpec((H, H), lambda i: (0, 0)),
            pl.BlockSpec((1, H), lambda i: (0, 0)),
        ],
        out_specs=pl.BlockSpec((N_BLK, H), lambda i: (i, 0)),
        out_shape=jax.ShapeDtypeStruct((N, H), jnp.float32),
    )(p, x, wa1, ba1r, wa2, ba2r)


def kernel(x, edge_index, edge_attr, W1, b1, W2, b2, W3, b3, Wa1, ba1, Wa2, ba2):
    src = edge_index[0].astype(jnp.int32)
    dst = edge_index[1].astype(jnp.int32)
    pad = E_PAD - E
    src_pad = jnp.concatenate([src, jnp.zeros((pad,), jnp.int32)])
    dst_pad = jnp.concatenate([dst, jnp.full((pad,), N, jnp.int32)])
    ea_pad = jnp.concatenate([edge_attr, jnp.zeros((pad, 1), jnp.float32)])
    x_bf = x.astype(jnp.bfloat16)

    w1p = jnp.zeros((64, FD), jnp.float32).at[:G].set(W1)
    smat = (jnp.arange(H * H)[:, None] // H == jnp.arange(H)[None, :]
            ).astype(jnp.bfloat16)

    xj = _sc_gather(x_bf, src_pad.reshape(NW, NCHUNK, CHUNK))
    msg = _msg_pallas(ea_pad, xj, w1p, b1.reshape(1, FD), W2,
                      b2.reshape(1, FD), W3.astype(jnp.bfloat16),
                      b3.reshape(H, H).T.astype(jnp.bfloat16), smat)
    partials = _sc_scatter(msg.reshape(NW, NCHUNK, CHUNK, H),
                           dst_pad.reshape(NW, NCHUNK, CHUNK),
                           jnp.zeros((ROWS_PER_TILE, H), jnp.bfloat16))
    return _ffn_pallas(partials, x, Wa1,
                       ba1.reshape(1, H), Wa2, ba2.reshape(1, H))
